# baseline (speedup 1.0000x reference)
import jax
import jax.numpy as jnp
import numpy as np
from jax import lax
from jax.experimental import pallas as pl
from jax.experimental.pallas import tpu as pltpu
from jax.experimental.pallas import tpu_sc as plsc

R = 128
N = 32768
K = 65
L = 16
NV = N // L
NC = 2
NS = 16
NW = NC * NS
ROWS_PER_W = R // NW
PIVOT = np.float32(2.5)
SIGN = np.int32(-(2**31))
LOW31 = np.int32(0x7FFFFFFF)
UNROLL = 8
CAP = 512
CV = CAP // L
CAPR = CV
NEG_INF = np.float32("-inf")


def _ckey(v):
    b = lax.bitcast_convert_type(v, jnp.int32)
    return jnp.where(b >= 0, b, b ^ LOW31)


def _descend(count_ge, first_bit=0, init=np.int32(0)):
    def bit_body(bi, p):
        bit = jnp.left_shift(jnp.int32(1), 31 - bi)
        cand_t = p | bit
        cs = cand_t ^ SIGN
        return jnp.where(count_ge(cs) >= K, cand_t, p)

    p_u = lax.fori_loop(first_bit, 32, bit_body, jnp.int32(init))
    return p_u ^ SIGN


def _compact_step(row_v, cand_v, i, cnt):
    v = row_v[pl.ds(i * L, L)]
    m = v > PIVOT
    mi = m.astype(jnp.int32)
    pos = jnp.left_shift(jnp.minimum(cnt, jnp.full((L,), CAPR, jnp.int32)),
                         4) + lax.iota(jnp.int32, L)
    plsc.store_scatter(cand_v, [pos], v, mask=m)
    return cnt + mi


def _prefill(cand_v):
    @plsc.parallel_loop(0, CV, unroll=4)
    def _fill(i):
        cand_v[pl.ds(i * L, L)] = jnp.full((L,), NEG_INF, jnp.float32)


def _compact(row_v, cand_v):
    _prefill(cand_v)

    @plsc.parallel_loop(0, NV, unroll=UNROLL,
                        carry=jnp.zeros((L,), jnp.int32))
    def cntv(i, cnt):
        return _compact_step(row_v, cand_v, i, cnt)

    return cntv


def _mask_and_compact(prev_v, thr, cur_v, cand_v):
    _prefill(cand_v)

    @plsc.parallel_loop(0, NV, unroll=UNROLL,
                        carry=jnp.zeros((L,), jnp.int32))
    def cntv(i, cnt):
        pv = prev_v[pl.ds(i * L, L)]
        prev_v[pl.ds(i * L, L)] = jnp.where(pv > thr, pv, jnp.float32(0.0))
        return _compact_step(cur_v, cand_v, i, cnt)

    return cntv


def _mask_only(row_v, thr):
    @plsc.parallel_loop(0, NV, unroll=UNROLL)
    def _mask(i):
        v = row_v[pl.ds(i * L, L)]
        row_v[pl.ds(i * L, L)] = jnp.where(v > thr, v, jnp.float32(0.0))


def _select(row_v, cand_v, cntv):
    cnt = jnp.sum(cntv)
    maxlane = jnp.max(cntv)

    def fast_path(_):
        def count_ge(cs):
            acc = jnp.zeros((L,), jnp.int32)
            one = jnp.full((L,), 1, jnp.int32)
            zero = jnp.zeros((L,), jnp.int32)
            for i in range(CV):
                kv = _ckey(cand_v[pl.ds(i * L, L)])
                acc = acc + jnp.where(kv >= cs, one, zero)
            return jnp.sum(acc)

        return _descend(count_ge, first_bit=2, init=np.int32(-(2**30)))

    def slow_path(_):
        def count_ge(cs):
            def cbody(i, acc):
                kv = _ckey(row_v[pl.ds(i * L, L)])
                return acc + jnp.where(kv >= cs,
                                       jnp.full((L,), 1, jnp.int32),
                                       jnp.full((L,), 0, jnp.int32))

            accv = lax.fori_loop(0, NV, cbody, jnp.zeros((L,), jnp.int32))
            return jnp.sum(accv)

        return _descend(count_ge)

    in_window = jnp.logical_and(cnt >= K, maxlane <= CAPR)
    vkey = lax.cond(in_window, fast_path, slow_path, jnp.int32(0))

    bsplat = jnp.full((L,), vkey, jnp.int32)
    bsplat = jnp.where(bsplat >= 0, bsplat, bsplat ^ LOW31)
    return lax.bitcast_convert_type(bsplat, jnp.float32)


def _sc_body(x_hbm, out_hbm, row0_v, row1_v, row2_v, cand_v,
             in_sem0, in_sem1, in_sem2, out_sem0, out_sem1, out_sem2):
    wid = lax.axis_index("s") * NC + lax.axis_index("c")
    r0 = wid * ROWS_PER_W
    bufs = [row0_v, row1_v, row2_v]
    in_sems = [in_sem0, in_sem1, in_sem2]
    out_sems = [out_sem0, out_sem1, out_sem2]
    nb = len(bufs)

    copies_in = [None] * ROWS_PER_W
    copies_out = [None] * ROWS_PER_W
    out_waited = [False] * ROWS_PER_W
    for j in range(min(nb, ROWS_PER_W)):
        copies_in[j] = pltpu.async_copy(x_hbm.at[r0 + j], bufs[j],
                                        in_sems[j])
    thr = None
    for j in range(ROWS_PER_W):
        b = j % nb
        row_v = bufs[b]
        copies_in[j].wait()

        if j == 0:
            cntv = _compact(row_v, cand_v)
        else:
            prev_v = bufs[(j - 1) % nb]
            cntv = _mask_and_compact(prev_v, thr, row_v, cand_v)
            copies_out[j - 1] = pltpu.async_copy(
                prev_v, out_hbm.at[r0 + j - 1], out_sems[(j - 1) % nb])

        jn = j - 2 + nb
        if j >= 2 and jn < ROWS_PER_W:
            bp = (j - 2) % nb
            copies_out[j - 2].wait()
            out_waited[j - 2] = True
            copies_in[jn] = pltpu.async_copy(x_hbm.at[r0 + jn], bufs[bp],
                                             in_sems[bp])

        thr = _select(row_v, cand_v, cntv)

    last = ROWS_PER_W - 1
    _mask_only(bufs[last % nb], thr)
    copies_out[last] = pltpu.async_copy(bufs[last % nb],
                                        out_hbm.at[r0 + last],
                                        out_sems[last % nb])
    for j in range(ROWS_PER_W):
        if copies_out[j] is not None and not out_waited[j]:
            copies_out[j].wait()


@jax.jit
def _ksparse_sc(x):
    mesh = plsc.VectorSubcoreMesh(core_axis_name="c", subcore_axis_name="s")
    return pl.kernel(
        _sc_body,
        out_type=jax.ShapeDtypeStruct((R, N), jnp.float32),
        mesh=mesh,
        compiler_params=pltpu.CompilerParams(
            needs_layout_passes=False,
            disable_bounds_checks=True,
            disable_semaphore_checks=True,
        ),
        scratch_types=[
            pltpu.VMEM((N,), jnp.float32),
            pltpu.VMEM((N,), jnp.float32),
            pltpu.VMEM((N,), jnp.float32),
            pltpu.VMEM((CAP + 2 * L,), jnp.float32),
            pltpu.SemaphoreType.DMA,
            pltpu.SemaphoreType.DMA,
            pltpu.SemaphoreType.DMA,
            pltpu.SemaphoreType.DMA,
            pltpu.SemaphoreType.DMA,
            pltpu.SemaphoreType.DMA,
        ],
    )(x)


def kernel(inputs):
    return _ksparse_sc(inputs)


# speedup vs baseline: 1.0618x; 1.0618x over previous
import jax
import jax.numpy as jnp
import numpy as np
from jax import lax
from jax.experimental import pallas as pl
from jax.experimental.pallas import tpu as pltpu
from jax.experimental.pallas import tpu_sc as plsc

R = 128
N = 32768
K = 65
L = 16
NV = N // L
NC = 2
NS = 16
NW = NC * NS
ROWS_PER_W = R // NW
PIVOT = np.float32(2.5)
SIGN = np.int32(-(2**31))
LOW31 = np.int32(0x7FFFFFFF)
UNROLL = 8
CAP = 512
CV = CAP // L
CAPR = CV
NEG_INF = np.float32("-inf")


def _ckey(v):
    b = lax.bitcast_convert_type(v, jnp.int32)
    return jnp.where(b >= 0, b, b ^ LOW31)


def _descend(count_ge, first_bit=0, init=np.int32(0)):
    def bit_body(bi, p):
        bit = jnp.left_shift(jnp.int32(1), 31 - bi)
        cand_t = p | bit
        cs = cand_t ^ SIGN
        return jnp.where(count_ge(cs) >= K, cand_t, p)

    p_u = lax.fori_loop(first_bit, 32, bit_body, jnp.int32(init))
    return p_u ^ SIGN


def _compact_step(row_v, cand_v, i, cnts):
    v = row_v[pl.ds(i * L, L)]
    m = v > PIVOT
    pos = jnp.minimum(cnts, jnp.left_shift(jnp.full((L,), CAPR, jnp.int32), 4)
                      + lax.iota(jnp.int32, L))
    plsc.store_scatter(cand_v, [pos], v, mask=m)
    return cnts + jnp.where(m, jnp.full((L,), L, jnp.int32),
                            jnp.zeros((L,), jnp.int32))


def _prefill(cand_v):
    @plsc.parallel_loop(0, CV, unroll=4)
    def _fill(i):
        cand_v[pl.ds(i * L, L)] = jnp.full((L,), NEG_INF, jnp.float32)


def _compact(row_v, cand_v):
    _prefill(cand_v)

    @plsc.parallel_loop(0, NV, unroll=UNROLL,
                        carry=lax.iota(jnp.int32, L))
    def cnts(i, c):
        return _compact_step(row_v, cand_v, i, c)

    return cnts


def _mask_and_compact(prev_v, thr, cur_v, cand_v):
    _prefill(cand_v)

    @plsc.parallel_loop(0, NV, unroll=UNROLL,
                        carry=lax.iota(jnp.int32, L))
    def cnts(i, c):
        pv = prev_v[pl.ds(i * L, L)]
        prev_v[pl.ds(i * L, L)] = jnp.where(pv > thr, pv, jnp.float32(0.0))
        return _compact_step(cur_v, cand_v, i, c)

    return cnts


def _mask_only(row_v, thr):
    @plsc.parallel_loop(0, NV, unroll=UNROLL)
    def _mask(i):
        v = row_v[pl.ds(i * L, L)]
        row_v[pl.ds(i * L, L)] = jnp.where(v > thr, v, jnp.float32(0.0))


def _select(row_v, cand_v, cntv):
    lane_counts = jnp.right_shift(cntv - lax.iota(jnp.int32, L), 4)
    cnt = jnp.sum(lane_counts)
    maxlane = jnp.max(lane_counts)

    def fast_path(_):
        def count_ge(cs):
            acc = jnp.zeros((L,), jnp.int32)
            one = jnp.full((L,), 1, jnp.int32)
            zero = jnp.zeros((L,), jnp.int32)
            for i in range(CV):
                kv = _ckey(cand_v[pl.ds(i * L, L)])
                acc = acc + jnp.where(kv >= cs, one, zero)
            return jnp.sum(acc)

        return _descend(count_ge, first_bit=2, init=np.int32(-(2**30)))

    def slow_path(_):
        def count_ge(cs):
            def cbody(i, acc):
                kv = _ckey(row_v[pl.ds(i * L, L)])
                return acc + jnp.where(kv >= cs,
                                       jnp.full((L,), 1, jnp.int32),
                                       jnp.full((L,), 0, jnp.int32))

            accv = lax.fori_loop(0, NV, cbody, jnp.zeros((L,), jnp.int32))
            return jnp.sum(accv)

        return _descend(count_ge)

    in_window = jnp.logical_and(cnt >= K, maxlane <= CAPR)
    vkey = lax.cond(in_window, fast_path, slow_path, jnp.int32(0))

    bsplat = jnp.full((L,), vkey, jnp.int32)
    bsplat = jnp.where(bsplat >= 0, bsplat, bsplat ^ LOW31)
    return lax.bitcast_convert_type(bsplat, jnp.float32)


def _sc_body(x_hbm, out_hbm, row0_v, row1_v, row2_v, cand_v,
             in_sem0, in_sem1, in_sem2, out_sem0, out_sem1, out_sem2):
    wid = lax.axis_index("s") * NC + lax.axis_index("c")
    r0 = wid * ROWS_PER_W
    bufs = [row0_v, row1_v, row2_v]
    in_sems = [in_sem0, in_sem1, in_sem2]
    out_sems = [out_sem0, out_sem1, out_sem2]
    nb = len(bufs)

    copies_in = [None] * ROWS_PER_W
    copies_out = [None] * ROWS_PER_W
    out_waited = [False] * ROWS_PER_W
    for j in range(min(nb, ROWS_PER_W)):
        copies_in[j] = pltpu.async_copy(x_hbm.at[r0 + j], bufs[j],
                                        in_sems[j])
    thr = None
    for j in range(ROWS_PER_W):
        b = j % nb
        row_v = bufs[b]
        copies_in[j].wait()

        if j == 0:
            cntv = _compact(row_v, cand_v)
        else:
            prev_v = bufs[(j - 1) % nb]
            cntv = _mask_and_compact(prev_v, thr, row_v, cand_v)
            copies_out[j - 1] = pltpu.async_copy(
                prev_v, out_hbm.at[r0 + j - 1], out_sems[(j - 1) % nb])

        jn = j - 2 + nb
        if j >= 2 and jn < ROWS_PER_W:
            bp = (j - 2) % nb
            copies_out[j - 2].wait()
            out_waited[j - 2] = True
            copies_in[jn] = pltpu.async_copy(x_hbm.at[r0 + jn], bufs[bp],
                                             in_sems[bp])

        thr = _select(row_v, cand_v, cntv)

    last = ROWS_PER_W - 1
    _mask_only(bufs[last % nb], thr)
    copies_out[last] = pltpu.async_copy(bufs[last % nb],
                                        out_hbm.at[r0 + last],
                                        out_sems[last % nb])
    for j in range(ROWS_PER_W):
        if copies_out[j] is not None and not out_waited[j]:
            copies_out[j].wait()


@jax.jit
def _ksparse_sc(x):
    mesh = plsc.VectorSubcoreMesh(core_axis_name="c", subcore_axis_name="s")
    return pl.kernel(
        _sc_body,
        out_type=jax.ShapeDtypeStruct((R, N), jnp.float32),
        mesh=mesh,
        compiler_params=pltpu.CompilerParams(needs_layout_passes=False),
        scratch_types=[
            pltpu.VMEM((N,), jnp.float32),
            pltpu.VMEM((N,), jnp.float32),
            pltpu.VMEM((N,), jnp.float32),
            pltpu.VMEM((CAP + 2 * L,), jnp.float32),
            pltpu.SemaphoreType.DMA,
            pltpu.SemaphoreType.DMA,
            pltpu.SemaphoreType.DMA,
            pltpu.SemaphoreType.DMA,
            pltpu.SemaphoreType.DMA,
            pltpu.SemaphoreType.DMA,
        ],
    )(x)


def kernel(inputs):
    return _ksparse_sc(inputs)
